# Initial kernel scaffold; baseline (speedup 1.0000x reference)
import jax
import jax.numpy as jnp
from jax.experimental import pallas as pl


def kernel(params, videos, edges, inputs):
    raise NotImplementedError("write your pallas kernel here")


# speedup vs baseline: 1.2638x; 1.2638x over previous
import functools

import jax
import jax.numpy as jnp
from jax.experimental import pallas as pl

N_USERS = 10000
H = 3
DH = 32
D = H * DH
TU = 400
TE = 1280
NEG = -1e30


def _matmul_kernel(x_ref, w_ref, b_ref, o_ref, *, act):
    o = jnp.dot(x_ref[...], w_ref[...], preferred_element_type=jnp.float32)
    o = o + b_ref[...]
    if act == "relu":
        o = jnp.maximum(o, 0.0)
    o_ref[...] = o


def _matmul(x, w, b, act=None, bm=2000):
    m, k = x.shape
    n = w.shape[1]
    mp = ((m + bm - 1) // bm) * bm
    if mp != m:
        x = jnp.pad(x, ((0, mp - m), (0, 0)))
    out = pl.pallas_call(
        functools.partial(_matmul_kernel, act=act),
        grid=(mp // bm,),
        in_specs=[
            pl.BlockSpec((bm, k), lambda i: (i, 0)),
            pl.BlockSpec((k, n), lambda i: (0, 0)),
            pl.BlockSpec((1, n), lambda i: (0, 0)),
        ],
        out_specs=pl.BlockSpec((bm, n), lambda i: (i, 0)),
        out_shape=jax.ShapeDtypeStruct((mp, n), jnp.float32),
    )(x, w, b.reshape(1, n))
    return out[:m]


def _pair_scores(qs, ms, mask_shape):
    pairs = []
    for h in range(H):
        p = qs[:, h][:, None] + ms[h, :][None, :]
        pairs.append(jnp.where(p > 0, p, 0.2 * p))
    return pairs


def _smax_kernel(qs_ref, ms_ref, src_ref, o_ref):
    u = pl.program_id(0)
    e = pl.program_id(1)
    uids = u * TU + jax.lax.broadcasted_iota(jnp.int32, (TU, TE), 0)
    mask = src_ref[...] == uids
    qs = qs_ref[...]
    ms = ms_ref[...]
    cols = []
    for p in _pair_scores(qs, ms, (TU, TE)):
        cols.append(jnp.where(mask, p, NEG).max(axis=1, keepdims=True))
    val = jnp.concatenate(cols, axis=1)

    @pl.when(e == 0)
    def _init():
        o_ref[...] = val

    @pl.when(e != 0)
    def _acc():
        o_ref[...] = jnp.maximum(o_ref[...], val)


def _ssum_kernel(qs_ref, smax_ref, ms_ref, src_ref, mh_ref, ssum_ref, num_ref):
    u = pl.program_id(0)
    e = pl.program_id(1)
    uids = u * TU + jax.lax.broadcasted_iota(jnp.int32, (TU, TE), 0)
    mask = src_ref[...] == uids
    qs = qs_ref[...]
    ms = ms_ref[...]
    smax = smax_ref[...]
    mh = mh_ref[...]
    ss_cols = []
    num_cols = []
    for h, p in enumerate(_pair_scores(qs, ms, (TU, TE))):
        ex = jnp.where(mask, jnp.exp(p - smax[:, h][:, None]), 0.0)
        ss_cols.append(ex.sum(axis=1, keepdims=True))
        num_cols.append(
            jnp.dot(ex, mh[:, h * DH:(h + 1) * DH],
                    preferred_element_type=jnp.float32))
    ssv = jnp.concatenate(ss_cols, axis=1)
    numv = jnp.concatenate(num_cols, axis=1)

    @pl.when(e == 0)
    def _init():
        ssum_ref[...] = ssv
        num_ref[...] = numv

    @pl.when(e != 0)
    def _acc():
        ssum_ref[...] += ssv
        num_ref[...] += numv


def _attention_layer(u, qs, ms_t, src2, m):
    nu = u.shape[0]
    ne = src2.shape[1]
    grid = (nu // TU, ne // TE)
    smax = pl.pallas_call(
        _smax_kernel,
        grid=grid,
        in_specs=[
            pl.BlockSpec((TU, H), lambda i, j: (i, 0)),
            pl.BlockSpec((H, TE), lambda i, j: (0, j)),
            pl.BlockSpec((1, TE), lambda i, j: (0, j)),
        ],
        out_specs=pl.BlockSpec((TU, H), lambda i, j: (i, 0)),
        out_shape=jax.ShapeDtypeStruct((nu, H), jnp.float32),
    )(qs, ms_t, src2)
    smax = jnp.where(smax <= NEG * 0.5, 0.0, smax)
    ssum, num = pl.pallas_call(
        _ssum_kernel,
        grid=grid,
        in_specs=[
            pl.BlockSpec((TU, H), lambda i, j: (i, 0)),
            pl.BlockSpec((TU, H), lambda i, j: (i, 0)),
            pl.BlockSpec((H, TE), lambda i, j: (0, j)),
            pl.BlockSpec((1, TE), lambda i, j: (0, j)),
            pl.BlockSpec((TE, D), lambda i, j: (j, 0)),
        ],
        out_specs=[
            pl.BlockSpec((TU, H), lambda i, j: (i, 0)),
            pl.BlockSpec((TU, D), lambda i, j: (i, 0)),
        ],
        out_shape=[
            jax.ShapeDtypeStruct((nu, H), jnp.float32),
            jax.ShapeDtypeStruct((nu, D), jnp.float32),
        ],
    )(qs, smax, ms_t, src2, m)
    msg = num / (jnp.repeat(ssum, DH, axis=1) + 1e-9)
    return u + jax.nn.elu(msg)


def kernel(params, videos, edges, inputs):
    t0, t1, t2, t3, t4, t5 = params["tables"]
    video_states = jnp.concatenate([t0[videos[:, 1]], t2[videos[:, 2]]], axis=1)
    edge_states = jnp.concatenate([
        params["user_embedding"][edges[:, 0]],
        t0[edges[:, 3]], t1[edges[:, 4]], t2[edges[:, 5]], t3[edges[:, 6]],
        t0[edges[:, 7]], t1[edges[:, 8]], t2[edges[:, 9]], t3[edges[:, 10]],
        t4[edges[:, 11]], t5[edges[:, 12]],
    ], axis=1)
    u = _matmul(params["user_embedding"], params["pre_u"]["W"],
                params["pre_u"]["b"], act="relu")
    v = _matmul(video_states, params["pre_v"]["W"], params["pre_v"]["b"],
                act="relu")
    e_proj = _matmul(edge_states, params["pre_e"]["W"], params["pre_e"]["b"],
                     act="relu")
    src_u = edges[:, 0]
    src_v = edges[:, 1]
    xm = jnp.concatenate([v[src_v], e_proj], axis=1)
    src2 = src_u.reshape(1, -1)
    zero_b = jnp.zeros((D,), jnp.float32)
    for p in params["att"]:
        qfull = _matmul(u, p["Wq"], zero_b)
        qs = (qfull.reshape(N_USERS, H, DH) * p["a"][None, :, :DH]).sum(-1)
        wcat = jnp.concatenate([p["Wv"], p["We"]], axis=0)
        m = _matmul(xm, wcat, zero_b)
        ms = (m.reshape(-1, H, DH) * p["a"][None, :, DH:]).sum(-1)
        u = _attention_layer(u, qs, ms.T, src2, m)
    u = params["bn_gamma"] * u / jnp.sqrt(1.0 + 0.001) + params["bn_beta"]
    out_all = _matmul(u, params["out"]["W"], params["out"]["b"])
    return out_all[inputs[:, 0]]
